# Initial kernel scaffold; baseline (speedup 1.0000x reference)
import jax
import jax.numpy as jnp
from jax.experimental import pallas as pl


def kernel(locations, cls, box, centerness, gt_boxes, gt_labels, gt_areas):
    raise NotImplementedError("write your pallas kernel here")


# speedup vs baseline: 1.3689x; 1.3689x over previous
import functools

import jax
import jax.numpy as jnp
from jax.experimental import pallas as pl
from jax.experimental.pallas import tpu as pltpu

INF = 1000000.0
LEVEL_SIZES = [12800, 3200, 800, 208, 56]
RANGES = [(-1.0, 64.0), (64.0, 128.0), (128.0, 256.0), (256.0, 512.0), (512.0, INF)]
P_TOTAL = sum(LEVEL_SIZES)
B = 4
G = 50
C = 80

PB = 1896
NB = P_TOTAL // PB


def _softplus_bce_dense(x):
    return jnp.maximum(x, 0.0) + jnp.log1p(jnp.exp(-jnp.abs(x)))


def _fcos_kernel(loc_ref, cls_ref, box_ref, ctr_ref, gtb_ref, gtl_ref, gta_ref,
                 out_ref, acc_ref):
    b = pl.program_id(0)
    k = pl.program_id(1)

    @pl.when((b == 0) & (k == 0))
    def _init():
        acc_ref[0] = 0.0
        acc_ref[1] = 0.0
        acc_ref[2] = 0.0
        acc_ref[3] = 0.0

    xs = loc_ref[:, 0:1]
    ys = loc_ref[:, 1:2]
    boxes = gtb_ref[0]
    bx0 = boxes[0:1, :]
    by0 = boxes[1:2, :]
    bx1 = boxes[2:3, :]
    by1 = boxes[3:4, :]

    l = xs - bx0
    t = ys - by0
    r = bx1 - xs
    bb = by1 - ys
    mn = jnp.minimum(jnp.minimum(l, t), jnp.minimum(r, bb))
    mx = jnp.maximum(jnp.maximum(l, t), jnp.maximum(r, bb))

    row = k * PB + jax.lax.broadcasted_iota(jnp.int32, (PB, 1), 0)
    o1, o2, o3, o4 = (LEVEL_SIZES[0],
                      LEVEL_SIZES[0] + LEVEL_SIZES[1],
                      LEVEL_SIZES[0] + LEVEL_SIZES[1] + LEVEL_SIZES[2],
                      LEVEL_SIZES[0] + LEVEL_SIZES[1] + LEVEL_SIZES[2] + LEVEL_SIZES[3])
    lo = jnp.where(row < o1, RANGES[0][0],
         jnp.where(row < o2, RANGES[1][0],
         jnp.where(row < o3, RANGES[2][0],
         jnp.where(row < o4, RANGES[3][0], RANGES[4][0]))))
    hi = jnp.where(row < o1, RANGES[0][1],
         jnp.where(row < o2, RANGES[1][1],
         jnp.where(row < o3, RANGES[2][1],
         jnp.where(row < o4, RANGES[3][1], RANGES[4][1]))))

    valid = (mn > 0.0) & (mn >= lo) & (mx <= hi)
    areas = jnp.where(valid, gta_ref[0], INF)
    min_area = jnp.min(areas, axis=1, keepdims=True)

    lane = jax.lax.broadcasted_iota(jnp.int32, (1, G), 1)
    min_ind = jnp.min(jnp.where(areas == min_area, lane, G), axis=1,
                      keepdims=True)
    onehot_g = lane == min_ind
    l_t = jnp.sum(jnp.where(onehot_g, l, 0.0), axis=1, keepdims=True)
    t_t = jnp.sum(jnp.where(onehot_g, t, 0.0), axis=1, keepdims=True)
    r_t = jnp.sum(jnp.where(onehot_g, r, 0.0), axis=1, keepdims=True)
    b_t = jnp.sum(jnp.where(onehot_g, bb, 0.0), axis=1, keepdims=True)
    lab = jnp.sum(jnp.where(onehot_g, gtl_ref[0], 0.0), axis=1, keepdims=True)
    lab = jnp.where(min_area >= INF, 0.0, lab)
    pos = lab > 0.0
    posf = pos.astype(jnp.float32)

    x = cls_ref[0]
    dense_sum = jnp.sum(_softplus_bce_dense(x))
    lane_c = jax.lax.broadcasted_iota(jnp.int32, (1, C), 1)
    onehot_c = (lane_c == (lab.astype(jnp.int32) - 1)) & pos
    pick_sum = jnp.sum(jnp.where(onehot_c, x, 0.0))
    cls_part = dense_sum - pick_sum

    tl = jnp.where(pos, l_t, 1.0)
    tt = jnp.where(pos, t_t, 1.0)
    tr = jnp.where(pos, r_t, 1.0)
    tb = jnp.where(pos, b_t, 1.0)
    ctr_tgt = ((jnp.minimum(tl, tr) / jnp.maximum(jnp.maximum(tl, tr), 1e-6))
               * (jnp.minimum(tt, tb) / jnp.maximum(jnp.maximum(tt, tb), 1e-6)))

    bx = box_ref[0]
    p_l = jnp.where(pos, jnp.maximum(bx[:, 0:1], 0.0), 1.0)
    p_t = jnp.where(pos, jnp.maximum(bx[:, 1:2], 0.0), 1.0)
    p_r = jnp.where(pos, jnp.maximum(bx[:, 2:3], 0.0), 1.0)
    p_b = jnp.where(pos, jnp.maximum(bx[:, 3:4], 0.0), 1.0)
    t_area = (tl + tr) * (tt + tb)
    p_area = (p_l + p_r) * (p_t + p_b)
    w_int = jnp.minimum(p_l, tl) + jnp.minimum(p_r, tr)
    h_int = jnp.minimum(p_t, tt) + jnp.minimum(p_b, tb)
    a_int = w_int * h_int
    a_union = t_area + p_area - a_int
    ious = (a_int + 1.0) / (a_union + 1.0)
    iou_l = -jnp.log(jnp.maximum(ious, 1e-6))
    w = ctr_tgt * posf
    iou_part = jnp.sum(iou_l * w)
    w_part = jnp.sum(w)

    cf = ctr_ref[0]
    ctr_bce = (jnp.maximum(cf, 0.0) - cf * ctr_tgt
               + jnp.log1p(jnp.exp(-jnp.abs(cf))))
    ctr_part = jnp.sum(ctr_bce * posf)

    acc_ref[0] += cls_part
    acc_ref[1] += iou_part
    acc_ref[2] += w_part
    acc_ref[3] += ctr_part

    @pl.when((b == B - 1) & (k == NB - 1))
    def _fin():
        lane_o = jax.lax.broadcasted_iota(jnp.int32, (1, 128), 1)
        cls_loss = acc_ref[0] * (1.0 / (B * P_TOTAL * C))
        reg_loss = acc_ref[1] / jnp.maximum(acc_ref[2], 1e-6)
        center_loss = acc_ref[3]
        out_ref[...] = (jnp.where(lane_o == 0, cls_loss, 0.0)
                        + jnp.where(lane_o == 1, reg_loss, 0.0)
                        + jnp.where(lane_o == 2, center_loss, 0.0))


@functools.partial(jax.jit, static_argnames=("interpret",))
def _run(locations, cls, box, centerness, gt_boxes, gt_labels, gt_areas,
         interpret=False):
    gtb_t = jnp.transpose(gt_boxes, (0, 2, 1))
    gtl = gt_labels.astype(jnp.float32)[:, None, :]
    gta = gt_areas[:, None, :]
    ctr3 = centerness[:, :, None]

    out = pl.pallas_call(
        _fcos_kernel,
        grid=(B, NB),
        in_specs=[
            pl.BlockSpec((PB, 2), lambda b, k: (k, 0)),
            pl.BlockSpec((1, PB, C), lambda b, k: (b, k, 0)),
            pl.BlockSpec((1, PB, 4), lambda b, k: (b, k, 0)),
            pl.BlockSpec((1, PB, 1), lambda b, k: (b, k, 0)),
            pl.BlockSpec((1, 4, G), lambda b, k: (b, 0, 0)),
            pl.BlockSpec((1, 1, G), lambda b, k: (b, 0, 0)),
            pl.BlockSpec((1, 1, G), lambda b, k: (b, 0, 0)),
        ],
        out_specs=pl.BlockSpec((1, 128), lambda b, k: (0, 0)),
        out_shape=jax.ShapeDtypeStruct((1, 128), jnp.float32),
        scratch_shapes=[pltpu.SMEM((4,), jnp.float32)],
        compiler_params=pltpu.CompilerParams(
            dimension_semantics=("arbitrary", "arbitrary")),
        interpret=interpret,
    )(locations, cls, box, ctr3, gtb_t, gtl, gta)
    return out[0, 0], out[0, 1], out[0, 2]


def kernel(locations, cls, box, centerness, gt_boxes, gt_labels, gt_areas):
    return _run(locations, cls, box, centerness, gt_boxes, gt_labels, gt_areas)


# speedup vs baseline: 3.6623x; 2.6755x over previous
import functools

import jax
import jax.numpy as jnp
from jax.experimental import pallas as pl
from jax.experimental.pallas import tpu as pltpu

INF = 1000000.0
LEVEL_SIZES = [12800, 3200, 800, 208, 56]
RANGES = [(-1.0, 64.0), (64.0, 128.0), (128.0, 256.0), (256.0, 512.0), (512.0, INF)]
P_TOTAL = sum(LEVEL_SIZES)
B = 4
G = 50
G_PAD = 56
C = 80

PBL = 1024
NB = -(-P_TOTAL // PBL)


def _fcos_kernel(loc_ref, cls_ref, box_ref, ctr_ref, gtb_ref, gtl_ref, gta_ref,
                 out_ref, acc_ref):
    b = pl.program_id(0)
    k = pl.program_id(1)

    @pl.when((b == 0) & (k == 0))
    def _init():
        acc_ref[0] = 0.0
        acc_ref[1] = 0.0
        acc_ref[2] = 0.0
        acc_ref[3] = 0.0

    xs = loc_ref[0:1, :]
    ys = loc_ref[1:2, :]
    gtb = gtb_ref[0]
    bx0 = gtb[:, 0:1]
    by0 = gtb[:, 1:2]
    bx1 = gtb[:, 2:3]
    by1 = gtb[:, 3:4]

    l = xs - bx0
    t = ys - by0
    r = bx1 - xs
    bb = by1 - ys
    mn = jnp.minimum(jnp.minimum(l, t), jnp.minimum(r, bb))
    mx = jnp.maximum(jnp.maximum(l, t), jnp.maximum(r, bb))

    pt = k * PBL + jax.lax.broadcasted_iota(jnp.int32, (1, PBL), 1)
    o1, o2, o3, o4 = (LEVEL_SIZES[0],
                      LEVEL_SIZES[0] + LEVEL_SIZES[1],
                      LEVEL_SIZES[0] + LEVEL_SIZES[1] + LEVEL_SIZES[2],
                      LEVEL_SIZES[0] + LEVEL_SIZES[1] + LEVEL_SIZES[2] + LEVEL_SIZES[3])
    lo = jnp.where(pt < o1, RANGES[0][0],
         jnp.where(pt < o2, RANGES[1][0],
         jnp.where(pt < o3, RANGES[2][0],
         jnp.where(pt < o4, RANGES[3][0], RANGES[4][0]))))
    hi = jnp.where(pt < o1, RANGES[0][1],
         jnp.where(pt < o2, RANGES[1][1],
         jnp.where(pt < o3, RANGES[2][1],
         jnp.where(pt < o4, RANGES[3][1], RANGES[4][1]))))

    valid = (mn > 0.0) & (mn >= lo) & (mx <= hi)
    areas = jnp.where(valid, gta_ref[0], INF)
    min_area = jnp.min(areas, axis=0, keepdims=True)

    g_iota = jax.lax.broadcasted_iota(jnp.int32, (G_PAD, 1), 0)
    min_ind = jnp.min(jnp.where(areas == min_area, g_iota, G_PAD), axis=0,
                      keepdims=True)
    onehot_g = g_iota == min_ind
    l_t = jnp.sum(jnp.where(onehot_g, l, 0.0), axis=0, keepdims=True)
    t_t = jnp.sum(jnp.where(onehot_g, t, 0.0), axis=0, keepdims=True)
    r_t = jnp.sum(jnp.where(onehot_g, r, 0.0), axis=0, keepdims=True)
    b_t = jnp.sum(jnp.where(onehot_g, bb, 0.0), axis=0, keepdims=True)

    pos = (min_area < INF) & (pt < P_TOTAL)

    a_mat = jnp.where(onehot_g & pos, 1.0, 0.0).astype(jnp.bfloat16)
    lane_c = jax.lax.broadcasted_iota(jnp.int32, (1, C), 1)
    m_mat = jnp.where(gtl_ref[0].astype(jnp.int32) == lane_c + 1,
                      1.0, 0.0).astype(jnp.bfloat16)
    w_cls = jax.lax.dot_general(a_mat, m_mat, (((0,), (0,)), ((), ())),
                                preferred_element_type=jnp.float32)
    x = cls_ref[0]
    col_pt = k * PBL + jax.lax.broadcasted_iota(jnp.int32, (PBL, 1), 0)
    col_valid = col_pt < P_TOTAL
    bce = jnp.maximum(x, 0.0) + jnp.log1p(jnp.exp(-jnp.abs(x))) - x * w_cls
    cls_part = jnp.sum(jnp.where(col_valid, bce, 0.0))

    tl = jnp.where(pos, l_t, 1.0)
    tt = jnp.where(pos, t_t, 1.0)
    tr = jnp.where(pos, r_t, 1.0)
    tb = jnp.where(pos, b_t, 1.0)
    ctr_tgt = ((jnp.minimum(tl, tr) / jnp.maximum(jnp.maximum(tl, tr), 1e-6))
               * (jnp.minimum(tt, tb) / jnp.maximum(jnp.maximum(tt, tb), 1e-6)))

    bx = box_ref[0]
    p_l = jnp.where(pos, jnp.maximum(bx[0:1, :], 0.0), 1.0)
    p_t = jnp.where(pos, jnp.maximum(bx[1:2, :], 0.0), 1.0)
    p_r = jnp.where(pos, jnp.maximum(bx[2:3, :], 0.0), 1.0)
    p_b = jnp.where(pos, jnp.maximum(bx[3:4, :], 0.0), 1.0)
    t_area = (tl + tr) * (tt + tb)
    p_area = (p_l + p_r) * (p_t + p_b)
    a_int = ((jnp.minimum(p_l, tl) + jnp.minimum(p_r, tr))
             * (jnp.minimum(p_t, tt) + jnp.minimum(p_b, tb)))
    a_union = t_area + p_area - a_int
    ious = (a_int + 1.0) / (a_union + 1.0)
    iou_l = -jnp.log(jnp.maximum(ious, 1e-6))
    iou_part = jnp.sum(jnp.where(pos, iou_l * ctr_tgt, 0.0))
    w_part = jnp.sum(jnp.where(pos, ctr_tgt, 0.0))

    cf = ctr_ref[0]
    ctr_bce = (jnp.maximum(cf, 0.0) - cf * ctr_tgt
               + jnp.log1p(jnp.exp(-jnp.abs(cf))))
    ctr_part = jnp.sum(jnp.where(pos, ctr_bce, 0.0))

    acc_ref[0] += cls_part
    acc_ref[1] += iou_part
    acc_ref[2] += w_part
    acc_ref[3] += ctr_part

    @pl.when((b == B - 1) & (k == NB - 1))
    def _fin():
        lane_o = jax.lax.broadcasted_iota(jnp.int32, (1, 128), 1)
        cls_loss = acc_ref[0] * (1.0 / (B * P_TOTAL * C))
        reg_loss = acc_ref[1] / jnp.maximum(acc_ref[2], 1e-6)
        center_loss = acc_ref[3]
        out_ref[...] = (jnp.where(lane_o == 0, cls_loss, 0.0)
                        + jnp.where(lane_o == 1, reg_loss, 0.0)
                        + jnp.where(lane_o == 2, center_loss, 0.0))


@functools.partial(jax.jit, static_argnames=("interpret",))
def _run(locations, cls, box, centerness, gt_boxes, gt_labels, gt_areas,
         interpret=False):
    loc_t = jnp.transpose(locations, (1, 0))
    box_t = jnp.transpose(box, (0, 2, 1))
    ctr3 = centerness[:, None, :]
    pad = [(0, 0), (0, G_PAD - G), (0, 0)]
    gtb = jnp.pad(gt_boxes, pad)
    gtl = jnp.pad(gt_labels.astype(jnp.float32), pad[:2])[:, :, None]
    gta = jnp.pad(gt_areas, pad[:2])[:, :, None]

    out = pl.pallas_call(
        _fcos_kernel,
        grid=(B, NB),
        in_specs=[
            pl.BlockSpec((2, PBL), lambda b, k: (0, k)),
            pl.BlockSpec((1, PBL, C), lambda b, k: (b, k, 0)),
            pl.BlockSpec((1, 4, PBL), lambda b, k: (b, 0, k)),
            pl.BlockSpec((1, 1, PBL), lambda b, k: (b, 0, k)),
            pl.BlockSpec((1, G_PAD, 4), lambda b, k: (b, 0, 0)),
            pl.BlockSpec((1, G_PAD, 1), lambda b, k: (b, 0, 0)),
            pl.BlockSpec((1, G_PAD, 1), lambda b, k: (b, 0, 0)),
        ],
        out_specs=pl.BlockSpec((1, 128), lambda b, k: (0, 0)),
        out_shape=jax.ShapeDtypeStruct((1, 128), jnp.float32),
        scratch_shapes=[pltpu.SMEM((4,), jnp.float32)],
        compiler_params=pltpu.CompilerParams(
            dimension_semantics=("arbitrary", "arbitrary")),
        interpret=interpret,
    )(loc_t, cls, box_t, ctr3, gtb, gtl, gta)
    return out[0, 0], out[0, 1], out[0, 2]


def kernel(locations, cls, box, centerness, gt_boxes, gt_labels, gt_areas):
    return _run(locations, cls, box, centerness, gt_boxes, gt_labels, gt_areas)


# speedup vs baseline: 4.4426x; 1.2131x over previous
import functools

import jax
import jax.numpy as jnp
from jax.experimental import pallas as pl
from jax.experimental.pallas import tpu as pltpu

INF = 1000000.0
LEVEL_SIZES = [12800, 3200, 800, 208, 56]
RANGES = [(-1.0, 64.0), (64.0, 128.0), (128.0, 256.0), (256.0, 512.0), (512.0, INF)]
P_TOTAL = sum(LEVEL_SIZES)
B = 4
G = 50
G_PAD = 56
C = 80

PBL = 2048
NB = -(-P_TOTAL // PBL)


def _fcos_kernel(loc_ref, cls_ref, box_ref, ctr_ref, gtb_ref, gtl_ref, gta_ref,
                 out_ref, acc_ref):
    k = pl.program_id(0)

    @pl.when(k == 0)
    def _init():
        acc_ref[0] = 0.0
        acc_ref[1] = 0.0
        acc_ref[2] = 0.0
        acc_ref[3] = 0.0

    xs = loc_ref[0:1, :]
    ys = loc_ref[1:2, :]
    pt = k * PBL + jax.lax.broadcasted_iota(jnp.int32, (1, PBL), 1)
    o1, o2, o3, o4 = (LEVEL_SIZES[0],
                      LEVEL_SIZES[0] + LEVEL_SIZES[1],
                      LEVEL_SIZES[0] + LEVEL_SIZES[1] + LEVEL_SIZES[2],
                      LEVEL_SIZES[0] + LEVEL_SIZES[1] + LEVEL_SIZES[2] + LEVEL_SIZES[3])
    lo = jnp.where(pt < o1, RANGES[0][0],
         jnp.where(pt < o2, RANGES[1][0],
         jnp.where(pt < o3, RANGES[2][0],
         jnp.where(pt < o4, RANGES[3][0], RANGES[4][0]))))
    hi = jnp.where(pt < o1, RANGES[0][1],
         jnp.where(pt < o2, RANGES[1][1],
         jnp.where(pt < o3, RANGES[2][1],
         jnp.where(pt < o4, RANGES[3][1], RANGES[4][1]))))
    in_range = pt < P_TOTAL
    col_valid = (k * PBL + jax.lax.broadcasted_iota(jnp.int32, (PBL, 1), 0)
                 ) < P_TOTAL
    g_iota = jax.lax.broadcasted_iota(jnp.int32, (G_PAD, 1), 0)
    lane_c = jax.lax.broadcasted_iota(jnp.int32, (1, C), 1)

    cls_acc = 0.0
    iou_acc = 0.0
    w_acc = 0.0
    ctr_acc = 0.0
    for i in range(B):
        gtb = gtb_ref[i]
        l = xs - gtb[:, 0:1]
        t = ys - gtb[:, 1:2]
        r = gtb[:, 2:3] - xs
        bb = gtb[:, 3:4] - ys
        mn = jnp.minimum(jnp.minimum(l, t), jnp.minimum(r, bb))
        mx = jnp.maximum(jnp.maximum(l, t), jnp.maximum(r, bb))
        valid = (mn > 0.0) & (mn >= lo) & (mx <= hi)
        areas = jnp.where(valid, gta_ref[i], INF)
        min_area = jnp.min(areas, axis=0, keepdims=True)
        min_ind = jnp.min(jnp.where(areas == min_area, g_iota, G_PAD),
                          axis=0, keepdims=True)
        pos = (min_area < INF) & in_range
        a_f32 = jnp.where((g_iota == min_ind) & pos, 1.0, 0.0)

        sel = jax.lax.dot_general(gtb, a_f32, (((0,), (0,)), ((), ())),
                                  preferred_element_type=jnp.float32)
        l_t = xs - sel[0:1, :]
        t_t = ys - sel[1:2, :]
        r_t = sel[2:3, :] - xs
        b_t = sel[3:4, :] - ys

        x = cls_ref[i]
        ax = jnp.abs(x)
        bce_d = (x + ax) * 0.5 + jnp.log1p(jnp.exp(-ax))
        dense_sum = jnp.sum(jnp.where(col_valid, bce_d, 0.0))
        x_bf = jnp.where(col_valid, x, 0.0).astype(jnp.bfloat16)
        picked = jax.lax.dot_general(a_f32.astype(jnp.bfloat16), x_bf,
                                     (((1,), (0,)), ((), ())),
                                     preferred_element_type=jnp.float32)
        m_sel = gtl_ref[i].astype(jnp.int32) == lane_c + 1
        pick_sum = jnp.sum(jnp.where(m_sel, picked, 0.0))
        cls_acc += dense_sum - pick_sum

        tl = jnp.where(pos, l_t, 1.0)
        tt = jnp.where(pos, t_t, 1.0)
        tr = jnp.where(pos, r_t, 1.0)
        tb = jnp.where(pos, b_t, 1.0)
        ctr_tgt = ((jnp.minimum(tl, tr) / jnp.maximum(jnp.maximum(tl, tr), 1e-6))
                   * (jnp.minimum(tt, tb) / jnp.maximum(jnp.maximum(tt, tb), 1e-6)))

        bx = box_ref[i]
        p_l = jnp.where(pos, jnp.maximum(bx[0:1, :], 0.0), 1.0)
        p_t = jnp.where(pos, jnp.maximum(bx[1:2, :], 0.0), 1.0)
        p_r = jnp.where(pos, jnp.maximum(bx[2:3, :], 0.0), 1.0)
        p_b = jnp.where(pos, jnp.maximum(bx[3:4, :], 0.0), 1.0)
        t_area = (tl + tr) * (tt + tb)
        p_area = (p_l + p_r) * (p_t + p_b)
        a_int = ((jnp.minimum(p_l, tl) + jnp.minimum(p_r, tr))
                 * (jnp.minimum(p_t, tt) + jnp.minimum(p_b, tb)))
        a_union = t_area + p_area - a_int
        ious = (a_int + 1.0) / (a_union + 1.0)
        iou_l = -jnp.log(jnp.maximum(ious, 1e-6))
        iou_acc += jnp.sum(jnp.where(pos, iou_l * ctr_tgt, 0.0))
        w_acc += jnp.sum(jnp.where(pos, ctr_tgt, 0.0))

        cf = ctr_ref[i]
        ctr_bce = (jnp.maximum(cf, 0.0) - cf * ctr_tgt
                   + jnp.log1p(jnp.exp(-jnp.abs(cf))))
        ctr_acc += jnp.sum(jnp.where(pos, ctr_bce, 0.0))

    acc_ref[0] += cls_acc
    acc_ref[1] += iou_acc
    acc_ref[2] += w_acc
    acc_ref[3] += ctr_acc

    @pl.when(k == NB - 1)
    def _fin():
        lane_o = jax.lax.broadcasted_iota(jnp.int32, (1, 128), 1)
        cls_loss = acc_ref[0] * (1.0 / (B * P_TOTAL * C))
        reg_loss = acc_ref[1] / jnp.maximum(acc_ref[2], 1e-6)
        center_loss = acc_ref[3]
        out_ref[...] = (jnp.where(lane_o == 0, cls_loss, 0.0)
                        + jnp.where(lane_o == 1, reg_loss, 0.0)
                        + jnp.where(lane_o == 2, center_loss, 0.0))


@functools.partial(jax.jit, static_argnames=("interpret",))
def _run(locations, cls, box, centerness, gt_boxes, gt_labels, gt_areas,
         interpret=False):
    loc_t = jnp.transpose(locations, (1, 0))
    box_t = jnp.transpose(box, (0, 2, 1))
    ctr3 = centerness[:, None, :]
    pad = [(0, 0), (0, G_PAD - G), (0, 0)]
    gtb = jnp.pad(gt_boxes, pad)
    gtl = jnp.pad(gt_labels.astype(jnp.float32), pad[:2])[:, :, None]
    gta = jnp.pad(gt_areas, pad[:2])[:, :, None]

    out = pl.pallas_call(
        _fcos_kernel,
        grid=(NB,),
        in_specs=[
            pl.BlockSpec((2, PBL), lambda k: (0, k)),
            pl.BlockSpec((B, PBL, C), lambda k: (0, k, 0)),
            pl.BlockSpec((B, 4, PBL), lambda k: (0, 0, k)),
            pl.BlockSpec((B, 1, PBL), lambda k: (0, 0, k)),
            pl.BlockSpec((B, G_PAD, 4), lambda k: (0, 0, 0)),
            pl.BlockSpec((B, G_PAD, 1), lambda k: (0, 0, 0)),
            pl.BlockSpec((B, G_PAD, 1), lambda k: (0, 0, 0)),
        ],
        out_specs=pl.BlockSpec((1, 128), lambda k: (0, 0)),
        out_shape=jax.ShapeDtypeStruct((1, 128), jnp.float32),
        scratch_shapes=[pltpu.SMEM((4,), jnp.float32)],
        compiler_params=pltpu.CompilerParams(
            dimension_semantics=("arbitrary",)),
        interpret=interpret,
    )(loc_t, cls, box_t, ctr3, gtb, gtl, gta)
    return out[0, 0], out[0, 1], out[0, 2]


def kernel(locations, cls, box, centerness, gt_boxes, gt_labels, gt_areas):
    return _run(locations, cls, box, centerness, gt_boxes, gt_labels, gt_areas)
